# baseline (speedup 1.0000x reference)
import functools

import jax
import jax.numpy as jnp
from jax import lax
from jax.experimental import pallas as pl
from jax.experimental.pallas import tpu as pltpu
from jax.experimental.pallas import tpu_sc as plsc

_NC = 2
_NS = 16
_NW = _NC * _NS


@functools.lru_cache(maxsize=None)
def _build_gather(N, V, D):
    n_per_w = N // _NW
    mesh = plsc.VectorSubcoreMesh(core_axis_name="c", subcore_axis_name="s")

    @functools.partial(
        pl.kernel,
        out_type=jax.ShapeDtypeStruct((N, D), jnp.float32),
        mesh=mesh,
        scratch_types=[
            pltpu.VMEM((n_per_w,), jnp.int32),
            pltpu.SemaphoreType.DMA,
        ],
    )
    def grab(idx_hbm, table_hbm, out_hbm, idx_s, sem):
        wid = lax.axis_index("s") * _NC + lax.axis_index("c")
        base = wid * n_per_w
        pltpu.sync_copy(idx_hbm.at[wid], idx_s)

        def group(j, carry):
            v = idx_s[pl.ds(j * 16, 16)]
            for l in range(16):
                pltpu.async_copy(
                    table_hbm.at[pl.ds(v[l], 1)],
                    out_hbm.at[pl.ds(base + j * 16 + l, 1)], sem)
            return carry

        lax.fori_loop(0, n_per_w // 16, group, 0)
        pltpu.make_async_copy(
            table_hbm.at[pl.ds(0, n_per_w)],
            out_hbm.at[pl.ds(base, n_per_w)], sem).wait()

    return grab


def kernel(positions, weight):
    B, S = positions.shape
    V, D = weight.shape
    N = B * S
    n_per_w = N // _NW
    idx = positions.astype(jnp.int32).reshape(_NW, n_per_w)
    out = _build_gather(N, V, D)(idx, weight)
    return out.reshape(B, S, D)


# speedup vs baseline: 36.1925x; 36.1925x over previous
import functools

import jax
import jax.numpy as jnp
from jax import lax
from jax.experimental import pallas as pl
from jax.experimental.pallas import tpu as pltpu
from jax.experimental.pallas import tpu_sc as plsc

_NC = 2
_NS = 16
_NW = _NC * _NS
_CHUNK = 32


@functools.lru_cache(maxsize=None)
def _build_gather(N, V, D):
    n_per_w = N // _NW
    n_chunks = n_per_w // _CHUNK
    assert n_chunks >= 2 and n_chunks % 2 == 0
    mesh = plsc.VectorSubcoreMesh(core_axis_name="c", subcore_axis_name="s")

    @functools.partial(
        pl.kernel,
        out_type=jax.ShapeDtypeStruct((N, D), jnp.float32),
        mesh=mesh,
        scratch_types=[
            pltpu.VMEM((n_chunks, _CHUNK), jnp.int32),
            pltpu.VMEM((_CHUNK, D), jnp.float32),
            pltpu.VMEM((_CHUNK, D), jnp.float32),
            pltpu.SemaphoreType.DMA,
            pltpu.SemaphoreType.DMA,
        ],
    )
    def grab(idx_hbm, table_hbm, out_hbm, idx_v, buf0, buf1, sem0, sem1):
        wid = lax.axis_index("s") * _NC + lax.axis_index("c")
        base = wid * n_per_w
        pltpu.sync_copy(idx_hbm.at[wid], idx_v)
        bufs = (buf0, buf1)
        sems = (sem0, sem1)

        def start(g, b):
            pltpu.async_copy(table_hbm.at[idx_v.at[g]], bufs[b], sems[b])

        def wait(g, b):
            pltpu.make_async_copy(
                table_hbm.at[idx_v.at[g]], bufs[b], sems[b]).wait()

        def write(g, b):
            pltpu.sync_copy(
                bufs[b], out_hbm.at[pl.ds(base + g * _CHUNK, _CHUNK)])

        start(0, 0)
        start(1, 1)

        def pair(p, carry):
            for b in range(2):
                g = 2 * p + b
                wait(g, b)
                write(g, b)
                start(g + 2, b)
            return carry

        lax.fori_loop(0, n_chunks // 2 - 1, pair, 0)
        for b in range(2):
            g = n_chunks - 2 + b
            wait(g, b)
            write(g, b)

    return grab


def kernel(positions, weight):
    B, S = positions.shape
    V, D = weight.shape
    N = B * S
    n_per_w = N // _NW
    idx = positions.astype(jnp.int32).reshape(_NW, n_per_w // _CHUNK, _CHUNK)
    out = _build_gather(N, V, D)(idx, weight)
    return out.reshape(B, S, D)
